# baseline (speedup 1.0000x reference)
import functools

import jax
import jax.numpy as jnp
from jax import lax
from jax.experimental import pallas as pl
from jax.experimental.pallas import tpu as pltpu
from jax.experimental.pallas import tpu_sc as plsc

N = 10000
F_IN = 256
H_DIM = 256
E = 160000

NC = 2
NS = 16
L = 16

NPAD = 10240
NSC = 10112
EPAD = 172032
EB = 128
ERT = EPAD // NS // EB
ERW = EPAD // (NC * NS) // EB
NT = NPAD // NS
NTS = NSC // NS
NCHUNK = 6
CW = 128


@functools.lru_cache(maxsize=1)
def _mesh():
    return plsc.VectorSubcoreMesh(core_axis_name="c", subcore_axis_name="s")


def _rsqrt16(v):
    i = lax.bitcast_convert_type(v, jnp.int32)
    y = lax.bitcast_convert_type(jnp.int32(0x5F3759DF) - (i >> 1), jnp.float32)
    for _ in range(3):
        y = y * (1.5 - 0.5 * v * y * y)
    return y


def _k1_body(col_h, w_h, roww_h, ww_h, dinv_h, ep_h,
             colv, wv, zb, degt, dinvloc, dinvfull, rowv2, wv2, epv2,
             deg_sp, dinv_sp):
    c = lax.axis_index("c")
    s = lax.axis_index("s")

    pltpu.sync_copy(col_h.at[s], colv)
    pltpu.sync_copy(w_h.at[s], wv)

    zero = jnp.zeros((L,), jnp.float32)
    for i in range(NT // L):
        zb[pl.ds(i * L, L)] = zero
    pltpu.sync_copy(zb, deg_sp.at[pl.ds(s * NT, NT)])
    plsc.subcore_barrier()

    def _deg(r, _):
        pltpu.sync_copy(wv.at[r], deg_sp.at[colv.at[r]], add=True)
        return _

    lax.fori_loop(0, ERT, _deg, None)
    plsc.subcore_barrier()

    pltpu.sync_copy(deg_sp.at[pl.ds(s * NT, NT)], degt)
    for j in range(NT // L):
        dinvloc[pl.ds(j * L, L)] = _rsqrt16(degt[pl.ds(j * L, L)])
    pltpu.sync_copy(dinvloc, dinv_sp.at[pl.ds(s * NT, NT)])

    @pl.when(c == 0)
    def _():
        pltpu.sync_copy(dinvloc, dinv_h.at[pl.ds(s * NT, NT)])

    plsc.subcore_barrier()
    pltpu.sync_copy(dinv_sp, dinvfull)

    widx = s * NC + c
    pltpu.sync_copy(roww_h.at[widx], rowv2)
    pltpu.sync_copy(ww_h.at[widx], wv2)

    def _ep(r, _):
        for k in range(EB // L):
            rows16 = rowv2[r, pl.ds(k * L, L)]
            dv = plsc.load_gather(dinvfull, [rows16])
            epv2[r, pl.ds(k * L, L)] = wv2[r, pl.ds(k * L, L)] * dv
        return _

    lax.fori_loop(0, ERW, _ep, None)
    pltpu.sync_copy(epv2, ep_h.at[widx])


def _k1(col3, w3, roww, ww):
    return pl.kernel(
        _k1_body,
        out_type=(
            jax.ShapeDtypeStruct((NPAD,), jnp.float32),
            jax.ShapeDtypeStruct((NC * NS, ERW, EB), jnp.float32),
        ),
        mesh=_mesh(),
        compiler_params=pltpu.CompilerParams(needs_layout_passes=False),
        scratch_types=[
            pltpu.VMEM((ERT, EB), jnp.int32),
            pltpu.VMEM((ERT, EB), jnp.float32),
            pltpu.VMEM((NT,), jnp.float32),
            pltpu.VMEM((NT,), jnp.float32),
            pltpu.VMEM((NT,), jnp.float32),
            pltpu.VMEM((NPAD,), jnp.float32),
            pltpu.VMEM((ERW, EB), jnp.int32),
            pltpu.VMEM((ERW, EB), jnp.float32),
            pltpu.VMEM((ERW, EB), jnp.float32),
            pltpu.VMEM_SHARED((NPAD,), jnp.float32),
            pltpu.VMEM_SHARED((NPAD,), jnp.float32),
        ],
    )(col3, w3, roww, ww)


def _k2_body(x_ref, w_ref, out_ref):
    xb = x_ref[...]
    for ci in range(NCHUNK):
        out_ref[ci] = jnp.dot(xb, w_ref[ci],
                              preferred_element_type=jnp.float32)


def _k2(x, w3c):
    bm = 1000
    return pl.pallas_call(
        _k2_body,
        grid=(N // bm,),
        in_specs=[
            pl.BlockSpec((bm, F_IN), lambda i: (i, 0)),
            pl.BlockSpec((NCHUNK, F_IN, CW), lambda i: (0, 0, 0)),
        ],
        out_specs=pl.BlockSpec((NCHUNK, bm, CW), lambda i: (0, i, 0)),
        out_shape=jax.ShapeDtypeStruct((NCHUNK, N, CW), jnp.float32),
    )(x, w3c)


def _k3_body(xw_h, col_h, row_h, ep_h, out_h,
             epv, rbuf, cbuf, gbufs, accs, gsem, isem):
    c = lax.axis_index("c")
    s = lax.axis_index("s")

    pltpu.sync_copy(ep_h.at[s], epv)

    zero = jnp.zeros((L,), jnp.float32)
    npc = NCHUNK // NC

    def run_chunk(ci, _0):
        ch = c * npc + ci

        def gz(a, _z):
            for b in range(CW // L):
                gbufs[0, a, pl.ds(b * L, L)] = zero
            return _z

        lax.fori_loop(0, EB, gz, None)
        base = s * NTS
        for i in range(NTS // EB):
            pltpu.sync_copy(gbufs.at[0], accs.at[pl.ds(base + i * EB, EB)])
        rem = NTS % EB
        if rem:
            pltpu.sync_copy(gbufs.at[0].at[pl.ds(0, rem)],
                            accs.at[pl.ds(base + (NTS // EB) * EB, rem)])
        plsc.subcore_barrier()

        pltpu.sync_copy(row_h.at[s].at[0], rbuf.at[0])
        pltpu.sync_copy(col_h.at[s].at[0], cbuf.at[0])
        pltpu.async_copy(xw_h.at[ch].at[rbuf.at[0]], gbufs.at[0], gsem)
        pltpu.async_copy(row_h.at[s].at[1], rbuf.at[1], isem)
        pltpu.async_copy(col_h.at[s].at[1], cbuf.at[1], isem)

        def erow(r, _):
            cur = lax.rem(r, 2)
            nxt = 1 - cur
            pltpu.make_async_copy(xw_h.at[ch].at[pl.ds(0, EB)],
                                  gbufs.at[cur], gsem).wait()

            @pl.when(r + 1 < ERT)
            def _():
                pltpu.make_async_copy(row_h.at[s].at[0], rbuf.at[nxt],
                                      isem).wait()
                pltpu.make_async_copy(col_h.at[s].at[0], cbuf.at[nxt],
                                      isem).wait()
                pltpu.async_copy(xw_h.at[ch].at[rbuf.at[nxt]],
                                 gbufs.at[nxt], gsem)

            def scale(g, _2):
                ep16 = epv[r, pl.ds(g * L, L)]
                for l in range(L):
                    sc = jnp.broadcast_to(ep16[l], (L,))
                    e = g * L + l
                    for j in range(CW // L):
                        gbufs[cur, e, pl.ds(j * L, L)] = (
                            gbufs[cur, e, pl.ds(j * L, L)] * sc)
                return _2

            lax.fori_loop(0, EB // L, scale, None)

            pltpu.sync_copy(gbufs.at[cur], accs.at[cbuf.at[cur]], add=True)

            @pl.when(r + 2 < ERT)
            def _():
                pltpu.async_copy(row_h.at[s].at[r + 2], rbuf.at[cur], isem)
                pltpu.async_copy(col_h.at[s].at[r + 2], cbuf.at[cur], isem)
            return _

        lax.fori_loop(0, ERT, erow, None)
        plsc.subcore_barrier()
        pltpu.sync_copy(accs.at[pl.ds(s * NTS, NTS)],
                        out_h.at[ch].at[pl.ds(s * NTS, NTS)])
        return _0

    lax.fori_loop(0, npc, run_chunk, None)


def _k3(xw, col3, row3, ep3):
    return pl.kernel(
        _k3_body,
        out_type=jax.ShapeDtypeStruct((NCHUNK, NSC, CW), jnp.float32),
        mesh=_mesh(),
        compiler_params=pltpu.CompilerParams(needs_layout_passes=False),
        scratch_types=[
            pltpu.VMEM((ERT, EB), jnp.float32),
            pltpu.VMEM((2, EB), jnp.int32),
            pltpu.VMEM((2, EB), jnp.int32),
            pltpu.VMEM((2, EB, CW), jnp.float32),
            pltpu.VMEM_SHARED((NSC, CW), jnp.float32),
            pltpu.SemaphoreType.DMA,
            pltpu.SemaphoreType.DMA,
        ],
    )(xw, col3, row3, ep3)


def _k4_body(a_ref, dinv_r, h_r,
             lza, lzb2, lra, lrb2, lha, lhb2,
             bz_r, br_r, bh_r, lzbias, lrbias, lhbias, wout_r, bout_r,
             out_l, out_h):
    dv = dinv_r[...]
    h = h_r[...]
    cpg = NCHUNK // 3
    gate = lambda g: jnp.concatenate(
        [a_ref[g * cpg + i] for i in range(cpg)], axis=1)
    cz = gate(0) * dv + bz_r[...]
    cr = gate(1) * dv + br_r[...]
    ch = gate(2) * dv + bh_r[...]

    def mm(a, b):
        return jnp.dot(a, b, preferred_element_type=jnp.float32)

    z = jax.nn.sigmoid(mm(cz, lza[...]) + mm(h, lzb2[...]) + lzbias[...])
    r = jax.nn.sigmoid(mm(cr, lra[...]) + mm(h, lrb2[...]) + lrbias[...])
    ht = jnp.tanh(mm(ch, lha[...]) + mm(h * r, lhb2[...]) + lhbias[...])
    hout = z * h + (1.0 - z) * ht
    out_h[...] = hout
    out_l[...] = mm(hout, wout_r[...]) + bout_r[...]


def _k4(accs, dinv2, h_prev, lza, lzb2, lra, lrb2, lha, lhb2,
        bz2, br2, bh2, lzb, lrb, lhb, wout, bout2):
    bm = 1000
    full = lambda shp: pl.BlockSpec(shp, lambda i: tuple(0 for _ in shp))
    return pl.pallas_call(
        _k4_body,
        grid=(N // bm,),
        in_specs=[pl.BlockSpec((NCHUNK, bm, CW), lambda i: (0, i, 0))] + [
            pl.BlockSpec((bm, 1), lambda i: (i, 0)),
            pl.BlockSpec((bm, H_DIM), lambda i: (i, 0)),
            full((H_DIM, H_DIM)), full((H_DIM, H_DIM)), full((H_DIM, H_DIM)),
            full((H_DIM, H_DIM)), full((H_DIM, H_DIM)), full((H_DIM, H_DIM)),
            full((1, H_DIM)), full((1, H_DIM)), full((1, H_DIM)),
            full((1, H_DIM)), full((1, H_DIM)), full((1, H_DIM)),
            full((H_DIM, 1)), full((1, 1)),
        ],
        out_specs=(
            pl.BlockSpec((bm, 1), lambda i: (i, 0)),
            pl.BlockSpec((bm, H_DIM), lambda i: (i, 0)),
        ),
        out_shape=(
            jax.ShapeDtypeStruct((N, 1), jnp.float32),
            jax.ShapeDtypeStruct((N, H_DIM), jnp.float32),
        ),
    )(accs, dinv2, h_prev, lza, lzb2, lra, lrb2, lha, lhb2,
      bz2, br2, bh2, lzb, lrb, lhb, wout, bout2)


def kernel(x, edge_index, edge_weight, h_prev, Wz, bz, Wr, br, Wh, bh,
           LzW, Lzb, LrW, Lrb, LhW, Lhb, Wout, bout):
    pad = EPAD - E - N
    loop = jnp.arange(N, dtype=jnp.int32)
    zpad_i = jnp.zeros((pad,), jnp.int32)
    row = jnp.concatenate([edge_index[0].astype(jnp.int32), loop, zpad_i])
    col = jnp.concatenate([edge_index[1].astype(jnp.int32), loop, zpad_i])
    w = jnp.concatenate([edge_weight, jnp.ones((N,), jnp.float32),
                         jnp.zeros((pad,), jnp.float32)])

    w3 = jnp.concatenate([Wz, Wr, Wh], axis=1)
    w3c = w3.reshape(F_IN, NCHUNK, CW).transpose(1, 0, 2)

    col3 = col.reshape(NS, ERT, EB)
    row3 = row.reshape(NS, ERT, EB)
    w3d = w.reshape(NS, ERT, EB)
    roww = row.reshape(NC * NS, ERW, EB)
    ww = w.reshape(NC * NS, ERW, EB)

    dinv, epw = _k1(col3, w3d, roww, ww)
    xw = _k2(x, w3c)
    accs = _k3(xw, col3, row3, epw.reshape(NS, ERT, EB))

    dinv2 = dinv[:, None]
    logits2, h_t = _k4(
        accs, dinv2, h_prev,
        LzW[:H_DIM], LzW[H_DIM:], LrW[:H_DIM], LrW[H_DIM:],
        LhW[:H_DIM], LhW[H_DIM:],
        bz[None, :], br[None, :], bh[None, :],
        Lzb[None, :], Lrb[None, :], Lhb[None, :],
        Wout, bout[None, :],
    )
    return logits2.reshape(N), h_t


# speedup vs baseline: 1.0991x; 1.0991x over previous
import functools

import jax
import jax.numpy as jnp
from jax import lax
from jax.experimental import pallas as pl
from jax.experimental.pallas import tpu as pltpu
from jax.experimental.pallas import tpu_sc as plsc

N = 10000
F_IN = 256
H_DIM = 256
E = 160000

NC = 2
NS = 16
L = 16

NPAD = 10240
NSC = 10112
EPAD = 172032
EB = 128
ERT = EPAD // NS // EB
ERW = EPAD // (NC * NS) // EB
NT = NPAD // NS
NTS = NSC // NS
NCHUNK = 6
CW = 128


@functools.lru_cache(maxsize=1)
def _mesh():
    return plsc.VectorSubcoreMesh(core_axis_name="c", subcore_axis_name="s")


def _rsqrt16(v):
    i = lax.bitcast_convert_type(v, jnp.int32)
    y = lax.bitcast_convert_type(jnp.int32(0x5F3759DF) - (i >> 1), jnp.float32)
    for _ in range(3):
        y = y * (1.5 - 0.5 * v * y * y)
    return y


def _k1_body(col_h, w_h, roww_h, ww_h, dinv_h, ep_h,
             colv, wv, zb, degt, dinvloc, dinvfull, rowv2, wv2, epv2,
             deg_sp, dinv_sp):
    c = lax.axis_index("c")
    s = lax.axis_index("s")

    pltpu.sync_copy(col_h.at[s], colv)
    pltpu.sync_copy(w_h.at[s], wv)

    zero = jnp.zeros((L,), jnp.float32)
    for i in range(NT // L):
        zb[pl.ds(i * L, L)] = zero
    pltpu.sync_copy(zb, deg_sp.at[pl.ds(s * NT, NT)])
    plsc.subcore_barrier()

    def _deg(r, _):
        pltpu.sync_copy(wv.at[r], deg_sp.at[colv.at[r]], add=True)
        return _

    lax.fori_loop(0, ERT, _deg, None)
    plsc.subcore_barrier()

    pltpu.sync_copy(deg_sp.at[pl.ds(s * NT, NT)], degt)
    for j in range(NT // L):
        dinvloc[pl.ds(j * L, L)] = _rsqrt16(degt[pl.ds(j * L, L)])
    pltpu.sync_copy(dinvloc, dinv_sp.at[pl.ds(s * NT, NT)])

    @pl.when(c == 0)
    def _():
        pltpu.sync_copy(dinvloc, dinv_h.at[pl.ds(s * NT, NT)])

    plsc.subcore_barrier()
    pltpu.sync_copy(dinv_sp, dinvfull)

    widx = s * NC + c
    pltpu.sync_copy(roww_h.at[widx], rowv2)
    pltpu.sync_copy(ww_h.at[widx], wv2)

    def _ep(r, _):
        for k in range(EB // L):
            rows16 = rowv2[r, pl.ds(k * L, L)]
            dv = plsc.load_gather(dinvfull, [rows16])
            epv2[r, pl.ds(k * L, L)] = wv2[r, pl.ds(k * L, L)] * dv
        return _

    lax.fori_loop(0, ERW, _ep, None)
    pltpu.sync_copy(epv2, ep_h.at[widx])


def _k1(col3, w3, roww, ww):
    return pl.kernel(
        _k1_body,
        out_type=(
            jax.ShapeDtypeStruct((NPAD,), jnp.float32),
            jax.ShapeDtypeStruct((NC * NS, ERW, EB), jnp.float32),
        ),
        mesh=_mesh(),
        compiler_params=pltpu.CompilerParams(needs_layout_passes=False),
        scratch_types=[
            pltpu.VMEM((ERT, EB), jnp.int32),
            pltpu.VMEM((ERT, EB), jnp.float32),
            pltpu.VMEM((NT,), jnp.float32),
            pltpu.VMEM((NT,), jnp.float32),
            pltpu.VMEM((NT,), jnp.float32),
            pltpu.VMEM((NPAD,), jnp.float32),
            pltpu.VMEM((ERW, EB), jnp.int32),
            pltpu.VMEM((ERW, EB), jnp.float32),
            pltpu.VMEM((ERW, EB), jnp.float32),
            pltpu.VMEM_SHARED((NPAD,), jnp.float32),
            pltpu.VMEM_SHARED((NPAD,), jnp.float32),
        ],
    )(col3, w3, roww, ww)


def _k2_body(x_ref, w_ref, out_ref):
    xb = x_ref[...]
    for ci in range(NCHUNK):
        out_ref[ci] = jnp.dot(xb, w_ref[ci],
                              preferred_element_type=jnp.float32)


def _k2(x, w3c):
    bm = 1000
    return pl.pallas_call(
        _k2_body,
        grid=(N // bm,),
        in_specs=[
            pl.BlockSpec((bm, F_IN), lambda i: (i, 0)),
            pl.BlockSpec((NCHUNK, F_IN, CW), lambda i: (0, 0, 0)),
        ],
        out_specs=pl.BlockSpec((NCHUNK, bm, CW), lambda i: (0, i, 0)),
        out_shape=jax.ShapeDtypeStruct((NCHUNK, N, CW), jnp.float32),
    )(x, w3c)


def _k3_body(xw_h, pc_h, ep_h, out_h,
             pcv, epbuf, rbuf, cbuf, gbufs, accs, gsem, ssem, esem):
    c = lax.axis_index("c")
    s = lax.axis_index("s")

    pltpu.sync_copy(pc_h.at[s], pcv)

    zero = jnp.zeros((L,), jnp.float32)
    npc = NCHUNK // NC
    mask = jnp.full((L,), 16383, jnp.int32)

    def unpack_rows(r, dst, slot):
        for g in range(EB // L):
            v = pcv[r, pl.ds(g * L, L)]
            dst[slot, pl.ds(g * L, L)] = v & mask
        return None

    def wait_gather(buf):
        pltpu.make_async_copy(xw_h.at[0].at[pl.ds(0, EB)], buf, gsem).wait()

    def wait_scatter():
        pltpu.make_async_copy(xw_h.at[0].at[pl.ds(0, EB)],
                              accs.at[pl.ds(0, EB)], ssem).wait()

    def run_chunk(ci, _0):
        ch = c * npc + ci

        def gz(a, _z):
            for b in range(CW // L):
                gbufs[0, a, pl.ds(b * L, L)] = zero
            return _z

        lax.fori_loop(0, EB, gz, None)
        base = s * NTS
        for i in range(NTS // EB):
            pltpu.sync_copy(gbufs.at[0], accs.at[pl.ds(base + i * EB, EB)])
        rem = NTS % EB
        if rem:
            pltpu.sync_copy(gbufs.at[0].at[pl.ds(0, rem)],
                            accs.at[pl.ds(base + (NTS // EB) * EB, rem)])
        plsc.subcore_barrier()

        unpack_rows(0, rbuf, 0)
        pltpu.sync_copy(ep_h.at[s].at[0], epbuf.at[0])
        pltpu.async_copy(xw_h.at[ch].at[rbuf.at[0]], gbufs.at[0], gsem)

        def erow(r, _):
            cur = lax.rem(r, 2)
            nxt = 1 - cur

            @pl.when(r > 0)
            def _():
                wait_scatter()

            @pl.when(r + 1 < ERT)
            def _():
                unpack_rows(r + 1, rbuf, nxt)
                pltpu.async_copy(xw_h.at[ch].at[rbuf.at[nxt]],
                                 gbufs.at[nxt], gsem)
                pltpu.async_copy(ep_h.at[s].at[r + 1], epbuf.at[nxt], esem)

            wait_gather(gbufs.at[cur])

            @pl.when(r > 0)
            def _():
                pltpu.make_async_copy(ep_h.at[s].at[0], epbuf.at[cur],
                                      esem).wait()

            def scale(g, _2):
                ep16 = epbuf[cur, pl.ds(g * L, L)]
                v = pcv[r, pl.ds(g * L, L)]
                cbuf[cur, pl.ds(g * L, L)] = lax.shift_right_logical(v, 14)
                for l in range(L):
                    sc = jnp.broadcast_to(ep16[l], (L,))
                    e = g * L + l
                    for j in range(CW // L):
                        gbufs[cur, e, pl.ds(j * L, L)] = (
                            gbufs[cur, e, pl.ds(j * L, L)] * sc)
                return _2

            lax.fori_loop(0, EB // L, scale, None)
            pltpu.async_copy(gbufs.at[cur], accs.at[cbuf.at[cur]], ssem,
                             add=True)
            return _

        lax.fori_loop(0, ERT, erow, None)
        wait_scatter()
        plsc.subcore_barrier()
        pltpu.sync_copy(accs.at[pl.ds(s * NTS, NTS)],
                        out_h.at[ch].at[pl.ds(s * NTS, NTS)])
        return _0

    lax.fori_loop(0, npc, run_chunk, None)


def _k3(xw, pc3, ep3):
    return pl.kernel(
        _k3_body,
        out_type=jax.ShapeDtypeStruct((NCHUNK, NSC, CW), jnp.float32),
        mesh=_mesh(),
        compiler_params=pltpu.CompilerParams(needs_layout_passes=False),
        scratch_types=[
            pltpu.VMEM((ERT, EB), jnp.int32),
            pltpu.VMEM((2, EB), jnp.float32),
            pltpu.VMEM((2, EB), jnp.int32),
            pltpu.VMEM((2, EB), jnp.int32),
            pltpu.VMEM((2, EB, CW), jnp.float32),
            pltpu.VMEM_SHARED((NSC, CW), jnp.float32),
            pltpu.SemaphoreType.DMA,
            pltpu.SemaphoreType.DMA,
            pltpu.SemaphoreType.DMA,
        ],
    )(xw, pc3, ep3)


def _k4_body(a_ref, dinv_r, h_r,
             lza, lzb2, lra, lrb2, lha, lhb2,
             bz_r, br_r, bh_r, lzbias, lrbias, lhbias, wout_r, bout_r,
             out_l, out_h):
    dv = dinv_r[...]
    h = h_r[...]
    cpg = NCHUNK // 3
    gate = lambda g: jnp.concatenate(
        [a_ref[g * cpg + i] for i in range(cpg)], axis=1)
    cz = gate(0) * dv + bz_r[...]
    cr = gate(1) * dv + br_r[...]
    ch = gate(2) * dv + bh_r[...]

    def mm(a, b):
        return jnp.dot(a, b, preferred_element_type=jnp.float32)

    z = jax.nn.sigmoid(mm(cz, lza[...]) + mm(h, lzb2[...]) + lzbias[...])
    r = jax.nn.sigmoid(mm(cr, lra[...]) + mm(h, lrb2[...]) + lrbias[...])
    ht = jnp.tanh(mm(ch, lha[...]) + mm(h * r, lhb2[...]) + lhbias[...])
    hout = z * h + (1.0 - z) * ht
    out_h[...] = hout
    out_l[...] = mm(hout, wout_r[...]) + bout_r[...]


def _k4(accs, dinv2, h_prev, lza, lzb2, lra, lrb2, lha, lhb2,
        bz2, br2, bh2, lzb, lrb, lhb, wout, bout2):
    bm = 1000
    full = lambda shp: pl.BlockSpec(shp, lambda i: tuple(0 for _ in shp))
    return pl.pallas_call(
        _k4_body,
        grid=(N // bm,),
        in_specs=[pl.BlockSpec((NCHUNK, bm, CW), lambda i: (0, i, 0))] + [
            pl.BlockSpec((bm, 1), lambda i: (i, 0)),
            pl.BlockSpec((bm, H_DIM), lambda i: (i, 0)),
            full((H_DIM, H_DIM)), full((H_DIM, H_DIM)), full((H_DIM, H_DIM)),
            full((H_DIM, H_DIM)), full((H_DIM, H_DIM)), full((H_DIM, H_DIM)),
            full((1, H_DIM)), full((1, H_DIM)), full((1, H_DIM)),
            full((1, H_DIM)), full((1, H_DIM)), full((1, H_DIM)),
            full((H_DIM, 1)), full((1, 1)),
        ],
        out_specs=(
            pl.BlockSpec((bm, 1), lambda i: (i, 0)),
            pl.BlockSpec((bm, H_DIM), lambda i: (i, 0)),
        ),
        out_shape=(
            jax.ShapeDtypeStruct((N, 1), jnp.float32),
            jax.ShapeDtypeStruct((N, H_DIM), jnp.float32),
        ),
    )(accs, dinv2, h_prev, lza, lzb2, lra, lrb2, lha, lhb2,
      bz2, br2, bh2, lzb, lrb, lhb, wout, bout2)


def kernel(x, edge_index, edge_weight, h_prev, Wz, bz, Wr, br, Wh, bh,
           LzW, Lzb, LrW, Lrb, LhW, Lhb, Wout, bout):
    pad = EPAD - E - N
    loop = jnp.arange(N, dtype=jnp.int32)
    zpad_i = jnp.zeros((pad,), jnp.int32)
    row = jnp.concatenate([edge_index[0].astype(jnp.int32), loop, zpad_i])
    col = jnp.concatenate([edge_index[1].astype(jnp.int32), loop, zpad_i])
    w = jnp.concatenate([edge_weight, jnp.ones((N,), jnp.float32),
                         jnp.zeros((pad,), jnp.float32)])

    w3 = jnp.concatenate([Wz, Wr, Wh], axis=1)
    w3c = w3.reshape(F_IN, NCHUNK, CW).transpose(1, 0, 2)

    col3 = col.reshape(NS, ERT, EB)
    row3 = row.reshape(NS, ERT, EB)
    w3d = w.reshape(NS, ERT, EB)
    roww = row.reshape(NC * NS, ERW, EB)
    ww = w.reshape(NC * NS, ERW, EB)

    dinv, epw = _k1(col3, w3d, roww, ww)
    xw = _k2(x, w3c)
    pc3 = (col * 16384 + row).reshape(NS, ERT, EB)
    accs = _k3(xw, pc3, epw.reshape(NS, ERT, EB))

    dinv2 = dinv[:, None]
    logits2, h_t = _k4(
        accs, dinv2, h_prev,
        LzW[:H_DIM], LzW[H_DIM:], LrW[:H_DIM], LrW[H_DIM:],
        LhW[:H_DIM], LhW[H_DIM:],
        bz[None, :], br[None, :], bh[None, :],
        Lzb[None, :], Lrb[None, :], Lhb[None, :],
        Wout, bout[None, :],
    )
    return logits2.reshape(N), h_t


# speedup vs baseline: 1.6557x; 1.5063x over previous
import functools

import jax
import jax.numpy as jnp
from jax import lax
from jax.experimental import pallas as pl
from jax.experimental.pallas import tpu as pltpu
from jax.experimental.pallas import tpu_sc as plsc

N = 10000
F_IN = 256
H_DIM = 256
E = 160000

NC = 2
NS = 16
L = 16

NPAD = 10240
NSC = 10112
EPAD = 172032
EB = 128
ERT = EPAD // NS // EB
ERW = EPAD // (NC * NS) // EB
NT = NPAD // NS
NTS = NSC // NS
NCHUNK = 6
CW = 128


@functools.lru_cache(maxsize=1)
def _mesh():
    return plsc.VectorSubcoreMesh(core_axis_name="c", subcore_axis_name="s")


def _rsqrt16(v):
    i = lax.bitcast_convert_type(v, jnp.int32)
    y = lax.bitcast_convert_type(jnp.int32(0x5F3759DF) - (i >> 1), jnp.float32)
    for _ in range(3):
        y = y * (1.5 - 0.5 * v * y * y)
    return y


def _k1_body(col_h, w_h, roww_h, ww_h, dinv_h, ep_h,
             colv, wv, zb, degt, dinvloc, dinvfull, rowv2, wv2, epv2,
             deg_sp, dinv_sp):
    c = lax.axis_index("c")
    s = lax.axis_index("s")

    pltpu.sync_copy(col_h.at[s], colv)
    pltpu.sync_copy(w_h.at[s], wv)

    zero = jnp.zeros((L,), jnp.float32)
    for i in range(NT // L):
        zb[pl.ds(i * L, L)] = zero
    pltpu.sync_copy(zb, deg_sp.at[pl.ds(s * NT, NT)])
    plsc.subcore_barrier()

    def _deg(r, _):
        pltpu.sync_copy(wv.at[r], deg_sp.at[colv.at[r]], add=True)
        return _

    lax.fori_loop(0, ERT, _deg, None)
    plsc.subcore_barrier()

    pltpu.sync_copy(deg_sp.at[pl.ds(s * NT, NT)], degt)
    for j in range(NT // L):
        dinvloc[pl.ds(j * L, L)] = _rsqrt16(degt[pl.ds(j * L, L)])
    pltpu.sync_copy(dinvloc, dinv_sp.at[pl.ds(s * NT, NT)])

    @pl.when(c == 0)
    def _():
        pltpu.sync_copy(dinvloc, dinv_h.at[pl.ds(s * NT, NT)])

    plsc.subcore_barrier()
    pltpu.sync_copy(dinv_sp, dinvfull)

    widx = s * NC + c
    pltpu.sync_copy(roww_h.at[widx], rowv2)
    pltpu.sync_copy(ww_h.at[widx], wv2)

    def _ep(r, _):
        for k in range(EB // L):
            rows16 = rowv2[r, pl.ds(k * L, L)]
            dv = plsc.load_gather(dinvfull, [rows16])
            epv2[r, pl.ds(k * L, L)] = wv2[r, pl.ds(k * L, L)] * dv
        return _

    lax.fori_loop(0, ERW, _ep, None)
    pltpu.sync_copy(epv2, ep_h.at[widx])


def _k1(col3, w3, roww, ww):
    return pl.kernel(
        _k1_body,
        out_type=(
            jax.ShapeDtypeStruct((NPAD,), jnp.float32),
            jax.ShapeDtypeStruct((NC * NS, ERW, EB), jnp.float32),
        ),
        mesh=_mesh(),
        compiler_params=pltpu.CompilerParams(needs_layout_passes=False),
        scratch_types=[
            pltpu.VMEM((ERT, EB), jnp.int32),
            pltpu.VMEM((ERT, EB), jnp.float32),
            pltpu.VMEM((NT,), jnp.float32),
            pltpu.VMEM((NT,), jnp.float32),
            pltpu.VMEM((NT,), jnp.float32),
            pltpu.VMEM((NPAD,), jnp.float32),
            pltpu.VMEM((ERW, EB), jnp.int32),
            pltpu.VMEM((ERW, EB), jnp.float32),
            pltpu.VMEM((ERW, EB), jnp.float32),
            pltpu.VMEM_SHARED((NPAD,), jnp.float32),
            pltpu.VMEM_SHARED((NPAD,), jnp.float32),
        ],
    )(col3, w3, roww, ww)


def _k2_body(x_ref, w_ref, out_ref):
    xb = x_ref[...]
    for ci in range(NCHUNK):
        out_ref[ci] = jnp.dot(xb, w_ref[ci],
                              preferred_element_type=jnp.float32)


def _k2(x, w3c):
    bm = 1000
    return pl.pallas_call(
        _k2_body,
        grid=(N // bm,),
        in_specs=[
            pl.BlockSpec((bm, F_IN), lambda i: (i, 0)),
            pl.BlockSpec((NCHUNK, F_IN, CW), lambda i: (0, 0, 0)),
        ],
        out_specs=pl.BlockSpec((NCHUNK, bm, CW), lambda i: (0, i, 0)),
        out_shape=jax.ShapeDtypeStruct((NCHUNK, N, CW), jnp.float32),
    )(x, w3c)


SB = 256
HR = EPAD // NS // SB // 2


def _k3_body(xw_h, pc_h, ep_h, out_h,
             pcv, epv, rbuf, cbuf, gbuf, accs, sem):
    c = lax.axis_index("c")
    s = lax.axis_index("s")

    zero = jnp.zeros((L,), jnp.float32)
    npc = NCHUNK // NC
    mask = jnp.full((L,), 16383, jnp.int32)

    def run_chunk(ci, _0):
        ch = c * npc + ci

        def gz(a, _z):
            for b in range(CW // L):
                gbuf[a, pl.ds(b * L, L)] = zero
            return _z

        lax.fori_loop(0, SB, gz, None)
        base = s * NTS
        for i in range(NTS // SB):
            pltpu.sync_copy(gbuf, accs.at[pl.ds(base + i * SB, SB)])
        rem = NTS % SB
        if rem:
            pltpu.sync_copy(gbuf.at[pl.ds(0, rem)],
                            accs.at[pl.ds(base + (NTS // SB) * SB, rem)])
        plsc.subcore_barrier()

        def run_half(hf, _h):
            pltpu.sync_copy(pc_h.at[s].at[hf], pcv)
            pltpu.sync_copy(ep_h.at[s].at[hf], epv)

            def ebatch(r, _):
                for g in range(SB // L):
                    v = pcv[r, pl.ds(g * L, L)]
                    rbuf[pl.ds(g * L, L)] = v & mask
                    cbuf[pl.ds(g * L, L)] = lax.shift_right_logical(v, 14)
                pltpu.async_copy(xw_h.at[ch].at[rbuf], gbuf, sem).wait()

                def scale(g, _2):
                    ep16 = epv[r, pl.ds(g * L, L)]
                    for l in range(L):
                        sc = jnp.broadcast_to(ep16[l], (L,))
                        e = g * L + l
                        for j in range(CW // L):
                            gbuf[e, pl.ds(j * L, L)] = (
                                gbuf[e, pl.ds(j * L, L)] * sc)
                    return _2

                lax.fori_loop(0, SB // L, scale, None)
                pltpu.sync_copy(gbuf, accs.at[cbuf], add=True)
                return _

            lax.fori_loop(0, HR, ebatch, None)
            return _h

        lax.fori_loop(0, 2, run_half, None)
        plsc.subcore_barrier()
        pltpu.sync_copy(accs.at[pl.ds(s * NTS, NTS)],
                        out_h.at[ch].at[pl.ds(s * NTS, NTS)])
        return _0

    lax.fori_loop(0, npc, run_chunk, None)


def _k3(xw, pc4, ep4):
    return pl.kernel(
        _k3_body,
        out_type=jax.ShapeDtypeStruct((NCHUNK, NSC, CW), jnp.float32),
        mesh=_mesh(),
        compiler_params=pltpu.CompilerParams(needs_layout_passes=False),
        scratch_types=[
            pltpu.VMEM((HR, SB), jnp.int32),
            pltpu.VMEM((HR, SB), jnp.float32),
            pltpu.VMEM((SB,), jnp.int32),
            pltpu.VMEM((SB,), jnp.int32),
            pltpu.VMEM((SB, CW), jnp.float32),
            pltpu.VMEM_SHARED((NSC, CW), jnp.float32),
            pltpu.SemaphoreType.DMA,
        ],
    )(xw, pc4, ep4)


def _k4_body(a_ref, dinv_r, h_r,
             lza, lzb2, lra, lrb2, lha, lhb2,
             bz_r, br_r, bh_r, lzbias, lrbias, lhbias, wout_r, bout_r,
             out_l, out_h):
    dv = dinv_r[...]
    h = h_r[...]
    cpg = NCHUNK // 3
    gate = lambda g: jnp.concatenate(
        [a_ref[g * cpg + i] for i in range(cpg)], axis=1)
    cz = gate(0) * dv + bz_r[...]
    cr = gate(1) * dv + br_r[...]
    ch = gate(2) * dv + bh_r[...]

    def mm(a, b):
        return jnp.dot(a, b, preferred_element_type=jnp.float32)

    z = jax.nn.sigmoid(mm(cz, lza[...]) + mm(h, lzb2[...]) + lzbias[...])
    r = jax.nn.sigmoid(mm(cr, lra[...]) + mm(h, lrb2[...]) + lrbias[...])
    ht = jnp.tanh(mm(ch, lha[...]) + mm(h * r, lhb2[...]) + lhbias[...])
    hout = z * h + (1.0 - z) * ht
    out_h[...] = hout
    out_l[...] = mm(hout, wout_r[...]) + bout_r[...]


def _k4(accs, dinv2, h_prev, lza, lzb2, lra, lrb2, lha, lhb2,
        bz2, br2, bh2, lzb, lrb, lhb, wout, bout2):
    bm = 1000
    full = lambda shp: pl.BlockSpec(shp, lambda i: tuple(0 for _ in shp))
    return pl.pallas_call(
        _k4_body,
        grid=(N // bm,),
        in_specs=[pl.BlockSpec((NCHUNK, bm, CW), lambda i: (0, i, 0))] + [
            pl.BlockSpec((bm, 1), lambda i: (i, 0)),
            pl.BlockSpec((bm, H_DIM), lambda i: (i, 0)),
            full((H_DIM, H_DIM)), full((H_DIM, H_DIM)), full((H_DIM, H_DIM)),
            full((H_DIM, H_DIM)), full((H_DIM, H_DIM)), full((H_DIM, H_DIM)),
            full((1, H_DIM)), full((1, H_DIM)), full((1, H_DIM)),
            full((1, H_DIM)), full((1, H_DIM)), full((1, H_DIM)),
            full((H_DIM, 1)), full((1, 1)),
        ],
        out_specs=(
            pl.BlockSpec((bm, 1), lambda i: (i, 0)),
            pl.BlockSpec((bm, H_DIM), lambda i: (i, 0)),
        ),
        out_shape=(
            jax.ShapeDtypeStruct((N, 1), jnp.float32),
            jax.ShapeDtypeStruct((N, H_DIM), jnp.float32),
        ),
    )(accs, dinv2, h_prev, lza, lzb2, lra, lrb2, lha, lhb2,
      bz2, br2, bh2, lzb, lrb, lhb, wout, bout2)


def kernel(x, edge_index, edge_weight, h_prev, Wz, bz, Wr, br, Wh, bh,
           LzW, Lzb, LrW, Lrb, LhW, Lhb, Wout, bout):
    pad = EPAD - E - N
    loop = jnp.arange(N, dtype=jnp.int32)
    zpad_i = jnp.zeros((pad,), jnp.int32)
    row = jnp.concatenate([edge_index[0].astype(jnp.int32), loop, zpad_i])
    col = jnp.concatenate([edge_index[1].astype(jnp.int32), loop, zpad_i])
    w = jnp.concatenate([edge_weight, jnp.ones((N,), jnp.float32),
                         jnp.zeros((pad,), jnp.float32)])

    w3 = jnp.concatenate([Wz, Wr, Wh], axis=1)
    w3c = w3.reshape(F_IN, NCHUNK, CW).transpose(1, 0, 2)

    col3 = col.reshape(NS, ERT, EB)
    row3 = row.reshape(NS, ERT, EB)
    w3d = w.reshape(NS, ERT, EB)
    roww = row.reshape(NC * NS, ERW, EB)
    ww = w.reshape(NC * NS, ERW, EB)

    dinv, epw = _k1(col3, w3d, roww, ww)
    xw = _k2(x, w3c)
    pc4 = (col * 16384 + row).reshape(NS, 2, HR, SB)
    accs = _k3(xw, pc4, epw.reshape(NS, 2, HR, SB))

    dinv2 = dinv[:, None]
    logits2, h_t = _k4(
        accs, dinv2, h_prev,
        LzW[:H_DIM], LzW[H_DIM:], LrW[:H_DIM], LrW[H_DIM:],
        LhW[:H_DIM], LhW[H_DIM:],
        bz[None, :], br[None, :], bh[None, :],
        Lzb[None, :], Lrb[None, :], Lhb[None, :],
        Wout, bout[None, :],
    )
    return logits2.reshape(N), h_t
